# Initial kernel scaffold; baseline (speedup 1.0000x reference)
import jax
import jax.numpy as jnp
from jax.experimental import pallas as pl


def kernel(x, p1, W, b, gamma, beta):
    raise NotImplementedError("write your pallas kernel here")


# speedup vs baseline: 7.9006x; 7.9006x over previous
import functools

import jax
import jax.numpy as jnp
from jax import lax
from jax.experimental import pallas as pl
from jax.experimental.pallas import tpu as pltpu

_N = 8192
_SAMP = 2048
_K = 16
_ROWS = 8
_COLS = _N // _ROWS
_BLK = 128
_NBLK = _SAMP // _BLK


def _fps_kernel(px_ref, py_ref, pz_ref, sel_ref):
    px = px_ref[...]
    py = py_ref[...]
    pz = pz_ref[...]
    iota = (lax.broadcasted_iota(jnp.int32, (_ROWS, _COLS), 0) * _COLS
            + lax.broadcasted_iota(jnp.int32, (_ROWS, _COLS), 1))
    sel_ref[0] = jnp.int32(0)
    lx0 = px_ref[0, 0]
    ly0 = py_ref[0, 0]
    lz0 = pz_ref[0, 0]
    dists0 = jnp.full((_ROWS, _COLS), jnp.inf, dtype=jnp.float32)

    def body(i, carry):
        dists, lx, ly, lz = carry
        dx = px - lx
        dy = py - ly
        dz = pz - lz
        d = dx * dx + dy * dy + dz * dz
        dists = jnp.minimum(dists, d)
        m = jnp.max(dists)
        eq = dists == m
        nxt = jnp.min(jnp.where(eq, iota, jnp.int32(1 << 30)))
        sel_ref[i] = nxt
        eqn = iota == nxt
        nlx = jnp.sum(jnp.where(eqn, px, 0.0))
        nly = jnp.sum(jnp.where(eqn, py, 0.0))
        nlz = jnp.sum(jnp.where(eqn, pz, 0.0))
        return (dists, nlx, nly, nlz)

    lax.fori_loop(1, _SAMP, body, (dists0, lx0, ly0, lz0))


def _knn_kernel(sx_ref, sy_ref, sz_ref, x_ref, y_ref, z_ref, out_ref):
    X = x_ref[...]
    Y = y_ref[...]
    Z = z_ref[...]
    sx = sx_ref[...]
    sy = sy_ref[...]
    sz = sz_ref[...]
    dx = X - sx
    dy = Y - sy
    dz = Z - sz
    D = dx * dx + dy * dy + dz * dz
    iota_n = lax.broadcasted_iota(jnp.int32, (_N, 1), 0)
    big = jnp.int32(1 << 30)
    inf = jnp.float32(jnp.inf)
    for t in range(_K + 1):
        m = jnp.min(D, axis=0, keepdims=True)
        cand = jnp.where(D == m, iota_n, big)
        idx = jnp.min(cand, axis=0, keepdims=True)
        out_ref[t, :] = idx[0]
        D = jnp.where(iota_n == idx, inf, D)


def _mlp_kernel(x_ref, w_ref, b_ref, g_ref, be_ref, out_ref):
    h = jnp.dot(x_ref[...], w_ref[...],
                preferred_element_type=jnp.float32) + b_ref[...]
    mu = jnp.mean(h, axis=0, keepdims=True)
    c = h - mu
    var = jnp.mean(c * c, axis=0, keepdims=True)
    hn = c / jnp.sqrt(var + 1e-5) * g_ref[...] + be_ref[...]
    out_ref[...] = jnp.maximum(hn, 0.0)


def _pool_kernel(nbr_ref, mlp_ref, out_ref):
    def body(s, _):
        acc = jnp.full((1, 512), -jnp.inf, dtype=jnp.float32)
        for t in range(1, _K + 1):
            n = nbr_ref[t, s]
            acc = jnp.maximum(acc, mlp_ref[pl.ds(n, 1), :])
        out_ref[pl.ds(s, 1), :] = acc
        return 0

    lax.fori_loop(0, _BLK, body, 0)


def kernel(x, p1, W, b, gamma, beta):
    px = p1[:, 0].reshape(_ROWS, _COLS)
    py = p1[:, 1].reshape(_ROWS, _COLS)
    pz = p1[:, 2].reshape(_ROWS, _COLS)

    sel = pl.pallas_call(
        _fps_kernel,
        out_shape=jax.ShapeDtypeStruct((_SAMP,), jnp.int32),
        out_specs=pl.BlockSpec(memory_space=pltpu.SMEM),
    )(px, py, pz)

    p2 = p1[sel]

    sx = p2[:, 0].reshape(1, _SAMP)
    sy = p2[:, 1].reshape(1, _SAMP)
    sz = p2[:, 2].reshape(1, _SAMP)
    Xc = p1[:, 0].reshape(_N, 1)
    Yc = p1[:, 1].reshape(_N, 1)
    Zc = p1[:, 2].reshape(_N, 1)

    samp_spec = pl.BlockSpec((1, _BLK), lambda i: (0, i))
    full_spec = pl.BlockSpec((_N, 1), lambda i: (0, 0))
    nbrs = pl.pallas_call(
        _knn_kernel,
        grid=(_NBLK,),
        in_specs=[samp_spec, samp_spec, samp_spec,
                  full_spec, full_spec, full_spec],
        out_specs=pl.BlockSpec((_K + 1, _BLK), lambda i: (0, i)),
        out_shape=jax.ShapeDtypeStruct((_K + 1, _SAMP), jnp.int32),
    )(sx, sy, sz, Xc, Yc, Zc)

    mlp = pl.pallas_call(
        _mlp_kernel,
        out_shape=jax.ShapeDtypeStruct((_N, 512), jnp.float32),
    )(x, W, b.reshape(1, 512), gamma.reshape(1, 512), beta.reshape(1, 512))

    y = pl.pallas_call(
        _pool_kernel,
        grid=(_NBLK,),
        in_specs=[
            pl.BlockSpec((_K + 1, _BLK), lambda i: (0, i),
                         memory_space=pltpu.SMEM),
            pl.BlockSpec((_N, 512), lambda i: (0, 0)),
        ],
        out_specs=pl.BlockSpec((_BLK, 512), lambda i: (i, 0)),
        out_shape=jax.ShapeDtypeStruct((_SAMP, 512), jnp.float32),
    )(nbrs, mlp)

    return (y, p2)


# speedup vs baseline: 9.4173x; 1.1920x over previous
import functools

import jax
import jax.numpy as jnp
from jax import lax
from jax.experimental import pallas as pl
from jax.experimental.pallas import tpu as pltpu

_N = 8192
_SAMP = 2048
_K = 16
_ROWS = 8
_COLS = _N // _ROWS
_BLK = 128
_NBLK = _SAMP // _BLK


def _fps_kernel(px_ref, py_ref, pz_ref, pxt_ref, pyt_ref, pzt_ref, sel_ref):
    px = px_ref[...]
    py = py_ref[...]
    pz = pz_ref[...]
    fiota = (lax.broadcasted_iota(jnp.int32, (_ROWS, _COLS), 0) * _COLS
             + lax.broadcasted_iota(jnp.int32, (_ROWS, _COLS), 1)
             ).astype(jnp.float32)
    sel_ref[0:1, 0:1] = jnp.zeros((1, 1), jnp.int32)
    lx0 = px[0:1, 0:1]
    ly0 = py[0:1, 0:1]
    lz0 = pz[0:1, 0:1]
    dists0 = jnp.full((_ROWS, _COLS), jnp.inf, dtype=jnp.float32)

    def body(i, carry):
        dists, lx, ly, lz = carry
        dx = px - lx
        dy = py - ly
        dz = pz - lz
        d = dx * dx + dy * dy + dz * dz
        dists = jnp.minimum(dists, d)
        m = jnp.max(dists, axis=(0, 1), keepdims=True)
        nxt = jnp.min(jnp.where(dists == m, fiota, jnp.float32(jnp.inf)),
                      axis=(0, 1), keepdims=True).astype(jnp.int32)
        sel_ref[pl.ds(i, 1), :] = nxt
        j = nxt[0, 0]
        nlx = pxt_ref[pl.ds(j, 1), :]
        nly = pyt_ref[pl.ds(j, 1), :]
        nlz = pzt_ref[pl.ds(j, 1), :]
        return (dists, nlx, nly, nlz)

    lax.fori_loop(1, _SAMP, body, (dists0, lx0, ly0, lz0))


def _knn_kernel(sx_ref, sy_ref, sz_ref, x_ref, y_ref, z_ref, out_ref):
    X = x_ref[...]
    Y = y_ref[...]
    Z = z_ref[...]
    sx = sx_ref[...]
    sy = sy_ref[...]
    sz = sz_ref[...]
    dx = X - sx
    dy = Y - sy
    dz = Z - sz
    D = dx * dx + dy * dy + dz * dz
    iota_n = lax.broadcasted_iota(jnp.int32, (_N, 1), 0)
    big = jnp.int32(1 << 30)
    inf = jnp.float32(jnp.inf)
    for t in range(_K + 1):
        m = jnp.min(D, axis=0, keepdims=True)
        cand = jnp.where(D == m, iota_n, big)
        idx = jnp.min(cand, axis=0, keepdims=True)
        out_ref[t, :] = idx[0]
        D = jnp.where(iota_n == idx, inf, D)


def _mlp_kernel(x_ref, w_ref, b_ref, g_ref, be_ref, out_ref):
    h = jnp.dot(x_ref[...], w_ref[...],
                preferred_element_type=jnp.float32) + b_ref[...]
    mu = jnp.mean(h, axis=0, keepdims=True)
    c = h - mu
    var = jnp.mean(c * c, axis=0, keepdims=True)
    hn = c / jnp.sqrt(var + 1e-5) * g_ref[...] + be_ref[...]
    out_ref[...] = jnp.maximum(hn, 0.0)


def _pool_kernel(nbr_ref, mlp_ref, out_ref):
    def body(s, _):
        acc = jnp.full((1, 512), -jnp.inf, dtype=jnp.float32)
        for t in range(1, _K + 1):
            n = nbr_ref[t, s]
            acc = jnp.maximum(acc, mlp_ref[pl.ds(n, 1), :])
        out_ref[pl.ds(s, 1), :] = acc
        return 0

    lax.fori_loop(0, _BLK, body, 0)


def kernel(x, p1, W, b, gamma, beta):
    px = p1[:, 0].reshape(_ROWS, _COLS)
    py = p1[:, 1].reshape(_ROWS, _COLS)
    pz = p1[:, 2].reshape(_ROWS, _COLS)

    sel2d = pl.pallas_call(
        _fps_kernel,
        out_shape=jax.ShapeDtypeStruct((_SAMP, 1), jnp.int32),
    )(px, py, pz,
      p1[:, 0].reshape(_N, 1), p1[:, 1].reshape(_N, 1),
      p1[:, 2].reshape(_N, 1))
    sel = sel2d[:, 0]

    p2 = p1[sel]

    sx = p2[:, 0].reshape(1, _SAMP)
    sy = p2[:, 1].reshape(1, _SAMP)
    sz = p2[:, 2].reshape(1, _SAMP)
    Xc = p1[:, 0].reshape(_N, 1)
    Yc = p1[:, 1].reshape(_N, 1)
    Zc = p1[:, 2].reshape(_N, 1)

    samp_spec = pl.BlockSpec((1, _BLK), lambda i: (0, i))
    full_spec = pl.BlockSpec((_N, 1), lambda i: (0, 0))
    nbrs = pl.pallas_call(
        _knn_kernel,
        grid=(_NBLK,),
        in_specs=[samp_spec, samp_spec, samp_spec,
                  full_spec, full_spec, full_spec],
        out_specs=pl.BlockSpec((_K + 1, _BLK), lambda i: (0, i)),
        out_shape=jax.ShapeDtypeStruct((_K + 1, _SAMP), jnp.int32),
    )(sx, sy, sz, Xc, Yc, Zc)

    mlp = pl.pallas_call(
        _mlp_kernel,
        out_shape=jax.ShapeDtypeStruct((_N, 512), jnp.float32),
    )(x, W, b.reshape(1, 512), gamma.reshape(1, 512), beta.reshape(1, 512))

    y = pl.pallas_call(
        _pool_kernel,
        grid=(_NBLK,),
        in_specs=[
            pl.BlockSpec((_K + 1, _BLK), lambda i: (0, i),
                         memory_space=pltpu.SMEM),
            pl.BlockSpec((_N, 512), lambda i: (0, 0)),
        ],
        out_specs=pl.BlockSpec((_BLK, 512), lambda i: (i, 0)),
        out_shape=jax.ShapeDtypeStruct((_SAMP, 512), jnp.float32),
    )(nbrs, mlp)

    return (y, p2)


# speedup vs baseline: 10.7629x; 1.1429x over previous
import functools

import jax
import jax.numpy as jnp
from jax import lax
from jax.experimental import pallas as pl
from jax.experimental.pallas import tpu as pltpu

_N = 8192
_SAMP = 2048
_K = 16
_ROWS = 8
_COLS = _N // _ROWS
_BLK = 128
_NBLK = _SAMP // _BLK


def _fps_kernel(px_ref, py_ref, pz_ref, pxt_ref, pyt_ref, pzt_ref, sel_ref):
    px = px_ref[...]
    py = py_ref[...]
    pz = pz_ref[...]
    fiota = (lax.broadcasted_iota(jnp.int32, (_ROWS, _COLS), 0) * _COLS
             + lax.broadcasted_iota(jnp.int32, (_ROWS, _COLS), 1)
             ).astype(jnp.float32)
    sel_ref[0:1, 0:1] = jnp.zeros((1, 1), jnp.int32)
    lx0 = px[0:1, 0:1]
    ly0 = py[0:1, 0:1]
    lz0 = pz[0:1, 0:1]
    dists0 = jnp.full((_ROWS, _COLS), jnp.inf, dtype=jnp.float32)

    def body(i, carry):
        dists, lx, ly, lz = carry
        dx = px - lx
        dy = py - ly
        dz = pz - lz
        d = dx * dx + dy * dy + dz * dz
        dists = jnp.minimum(dists, d)
        m = jnp.max(dists, axis=(0, 1), keepdims=True)
        nxt = jnp.min(jnp.where(dists == m, fiota, jnp.float32(jnp.inf)),
                      axis=(0, 1), keepdims=True).astype(jnp.int32)
        sel_ref[pl.ds(i, 1), :] = nxt
        j = nxt[0, 0]
        nlx = pxt_ref[pl.ds(j, 1), :]
        nly = pyt_ref[pl.ds(j, 1), :]
        nlz = pzt_ref[pl.ds(j, 1), :]
        return (dists, nlx, nly, nlz)

    lax.fori_loop(1, _SAMP, body, (dists0, lx0, ly0, lz0))


def _treemin(v):
    while v.shape[0] > 8:
        c = v.shape[0] // 8
        acc = jnp.minimum(v[:c], v[c:2 * c])
        for k in range(2, 8):
            acc = jnp.minimum(acc, v[k * c:(k + 1) * c])
        v = acc
    return jnp.min(v, axis=0, keepdims=True)


def _knn_kernel(sx_ref, sy_ref, sz_ref, x_ref, y_ref, z_ref, out_ref):
    X = x_ref[...]
    Y = y_ref[...]
    Z = z_ref[...]
    sx = sx_ref[...]
    sy = sy_ref[...]
    sz = sz_ref[...]
    dx = X - sx
    dy = Y - sy
    dz = Z - sz
    D = dx * dx + dy * dy + dz * dz
    fiota_n = lax.broadcasted_iota(jnp.int32, (_N, 1), 0).astype(jnp.float32)
    inf = jnp.float32(jnp.inf)
    for t in range(_K + 1):
        m = _treemin(D)
        cand = jnp.where(D == m, fiota_n, inf)
        idx = _treemin(cand)
        out_ref[t, :] = idx[0].astype(jnp.int32)
        D = jnp.where(fiota_n == idx, inf, D)


def _mlp_kernel(x_ref, w_ref, b_ref, g_ref, be_ref, out_ref):
    h = jnp.dot(x_ref[...], w_ref[...],
                preferred_element_type=jnp.float32) + b_ref[...]
    mu = jnp.mean(h, axis=0, keepdims=True)
    c = h - mu
    var = jnp.mean(c * c, axis=0, keepdims=True)
    hn = c / jnp.sqrt(var + 1e-5) * g_ref[...] + be_ref[...]
    out_ref[...] = jnp.maximum(hn, 0.0)


def _pool_kernel(nbr_ref, mlp_ref, out_ref):
    def body(s, _):
        acc = jnp.full((1, 512), -jnp.inf, dtype=jnp.float32)
        for t in range(1, _K + 1):
            n = nbr_ref[t, s]
            acc = jnp.maximum(acc, mlp_ref[pl.ds(n, 1), :])
        out_ref[pl.ds(s, 1), :] = acc
        return 0

    lax.fori_loop(0, _BLK, body, 0)


def kernel(x, p1, W, b, gamma, beta):
    px = p1[:, 0].reshape(_ROWS, _COLS)
    py = p1[:, 1].reshape(_ROWS, _COLS)
    pz = p1[:, 2].reshape(_ROWS, _COLS)

    sel2d = pl.pallas_call(
        _fps_kernel,
        out_shape=jax.ShapeDtypeStruct((_SAMP, 1), jnp.int32),
    )(px, py, pz,
      p1[:, 0].reshape(_N, 1), p1[:, 1].reshape(_N, 1),
      p1[:, 2].reshape(_N, 1))
    sel = sel2d[:, 0]

    p2 = p1[sel]

    sx = p2[:, 0].reshape(1, _SAMP)
    sy = p2[:, 1].reshape(1, _SAMP)
    sz = p2[:, 2].reshape(1, _SAMP)
    Xc = p1[:, 0].reshape(_N, 1)
    Yc = p1[:, 1].reshape(_N, 1)
    Zc = p1[:, 2].reshape(_N, 1)

    samp_spec = pl.BlockSpec((1, _BLK), lambda i: (0, i))
    full_spec = pl.BlockSpec((_N, 1), lambda i: (0, 0))
    nbrs = pl.pallas_call(
        _knn_kernel,
        grid=(_NBLK,),
        in_specs=[samp_spec, samp_spec, samp_spec,
                  full_spec, full_spec, full_spec],
        out_specs=pl.BlockSpec((_K + 1, _BLK), lambda i: (0, i)),
        out_shape=jax.ShapeDtypeStruct((_K + 1, _SAMP), jnp.int32),
    )(sx, sy, sz, Xc, Yc, Zc)

    mlp = pl.pallas_call(
        _mlp_kernel,
        out_shape=jax.ShapeDtypeStruct((_N, 512), jnp.float32),
    )(x, W, b.reshape(1, 512), gamma.reshape(1, 512), beta.reshape(1, 512))

    y = pl.pallas_call(
        _pool_kernel,
        grid=(_NBLK,),
        in_specs=[
            pl.BlockSpec((_K + 1, _BLK), lambda i: (0, i),
                         memory_space=pltpu.SMEM),
            pl.BlockSpec((_N, 512), lambda i: (0, 0)),
        ],
        out_specs=pl.BlockSpec((_BLK, 512), lambda i: (i, 0)),
        out_shape=jax.ShapeDtypeStruct((_SAMP, 512), jnp.float32),
    )(nbrs, mlp)

    return (y, p2)
